# Initial kernel scaffold; baseline (speedup 1.0000x reference)
import jax
import jax.numpy as jnp
from jax.experimental import pallas as pl


def kernel(f_atoms, f_bonds, a2a, a2b, W_i, b_i, W_h, b_h, W_o, b_o, W_f, b_f):
    raise NotImplementedError("write your pallas kernel here")


# speedup vs baseline: 1.0464x; 1.0464x over previous
import functools

import jax
import jax.numpy as jnp
from jax import lax
from jax.experimental import pallas as pl
from jax.experimental.pallas import tpu as pltpu
from jax.experimental.pallas import tpu_sc as plsc

N = 10000
NB = 32
H = 128
BF = 16
NMOL = 100
APM = 100
DEPTH = 3

NC = 2
NS = 16
NW = NC * NS
NP = 10240
AW = NP // NW
SA = 8
STEPS = AW // SA
CH = (SA * NB) // 128


def _reduce_rows(rows_v, out_v, d):
    nch = d // 16
    for a in range(SA):
        def body(j, accs, a=a):
            return tuple(
                accs[c] + rows_v[a * NB + j, pl.ds(c * 16, 16)]
                for c in range(nch))
        accs = lax.fori_loop(
            0, NB, body, tuple(jnp.zeros((16,), jnp.float32)
                               for _ in range(nch)))
        for c in range(nch):
            out_v[a, pl.ds(c * 16, 16)] = accs[c]


def _gather_sum_kernel(table_hbm, idx_hbm, out_hbm, idx_v, rows_v, out_v):
    c = lax.axis_index("c")
    s = lax.axis_index("s")
    w = c * NS + s
    d = out_v.shape[1]

    @pl.loop(0, STEPS)
    def _(t):
        pltpu.sync_copy(idx_hbm.at[w, t], idx_v)
        for j in range(CH):
            pltpu.sync_copy(table_hbm.at[idx_v.at[j]],
                            rows_v.at[pl.ds(j * 128, 128)])
        _reduce_rows(rows_v, out_v, d)
        pltpu.sync_copy(out_v, out_hbm.at[pl.ds(w * AW + t * SA, SA)])


def _gather_sum(table, idx4, d):
    mesh = plsc.VectorSubcoreMesh(core_axis_name="c", subcore_axis_name="s")
    return pl.kernel(
        _gather_sum_kernel,
        out_type=jax.ShapeDtypeStruct((NP, d), jnp.float32),
        mesh=mesh,
        scratch_types=[
            pltpu.VMEM((CH, 128), jnp.int32),
            pltpu.VMEM((SA * NB, d), jnp.float32),
            pltpu.VMEM((SA, d), jnp.float32),
        ],
    )(table, idx4)


BM = 256


def _pre_kernel(a_ref, wi_ref, bi_ref, wo_ref, x_ref, msg_ref, fo_ref):
    a = a_ref[...]
    x = jnp.dot(a, wi_ref[...], preferred_element_type=jnp.float32) + bi_ref[...]
    x_ref[...] = x
    msg_ref[...] = jnp.maximum(x, 0.0)
    fo_ref[...] = jnp.dot(a, wo_ref[...], preferred_element_type=jnp.float32)


def _bondproj_kernel(fb_ref, wb_ref, out_ref):
    out_ref[...] = jnp.dot(fb_ref[...], wb_ref[...],
                           preferred_element_type=jnp.float32)


def _update_kernel(x_ref, g_ref, cpart_ref, wa_ref, bh_ref, out_ref):
    m = jnp.dot(g_ref[...], wa_ref[...], preferred_element_type=jnp.float32)
    out_ref[...] = jnp.maximum(x_ref[...] + m + cpart_ref[...] + bh_ref[...],
                               0.0)


def _final_kernel(fo_ref, g_ref, wo2_ref, bo_ref, out_ref):
    m = jnp.dot(g_ref[...], wo2_ref[...], preferred_element_type=jnp.float32)
    out_ref[...] = jnp.maximum(fo_ref[...] + m + bo_ref[...], 0.0)


def _readout_kernel(ah_ref, out_ref):
    out_ref[0] = jnp.max(ah_ref[0], axis=0, keepdims=True)


def _ffn_kernel(mv_ref, wf_ref, bf_ref, out_ref):
    out_ref[...] = (jnp.dot(mv_ref[...], wf_ref[...],
                            preferred_element_type=jnp.float32) + bf_ref[0])


def _row_block(i):
    return (i, 0)


def _full(i):
    return (0, 0)


@jax.jit
def kernel(f_atoms, f_bonds, a2a, a2b, W_i, b_i, W_h, b_h, W_o, b_o, W_f, b_f):
    f32 = jnp.float32
    f_atoms = f_atoms.astype(f32)
    f_bonds = f_bonds.astype(f32)

    f_pad = jnp.pad(f_atoms, ((0, NP - N), (0, 0)))
    a2a_i = jnp.pad(a2a.astype(jnp.int32), ((0, NP - N), (0, 0)))
    a2b_i = jnp.pad(a2b.astype(jnp.int32), ((0, NP - N), (0, 0)))
    a2a4 = a2a_i.reshape(NW, STEPS, CH, 128)
    a2b4 = a2b_i.reshape(NW, STEPS, CH, 128)
    W_ha = W_h[:H]
    W_hb = W_h[H:]
    W_ot = W_o[:H]
    W_ob = W_o[H:]
    b_i2 = b_i.reshape(1, H)
    b_h2 = b_h.reshape(1, H)
    b_o2 = b_o.reshape(1, H)

    grid = NP // BM
    row_spec = pl.BlockSpec((BM, H), _row_block)
    w_spec = pl.BlockSpec((H, H), _full)
    wb_spec = pl.BlockSpec((BF, H), _full)
    b_spec = pl.BlockSpec((1, H), _full)
    out_rows = jax.ShapeDtypeStruct((NP, H), f32)

    x, msg, fo = pl.pallas_call(
        _pre_kernel,
        grid=(grid,),
        in_specs=[row_spec, w_spec, b_spec, w_spec],
        out_specs=[row_spec, row_spec, row_spec],
        out_shape=[out_rows, out_rows, out_rows],
    )(f_pad, W_i, b_i2, W_ot)

    BMB = 1280
    fbp = pl.pallas_call(
        _bondproj_kernel,
        grid=(f_bonds.shape[0] // BMB,),
        in_specs=[pl.BlockSpec((BMB, BF), _row_block), wb_spec],
        out_specs=pl.BlockSpec((BMB, H), _row_block),
        out_shape=jax.ShapeDtypeStruct((f_bonds.shape[0], H), f32),
    )(f_bonds, W_hb)
    cpart = _gather_sum(fbp, a2b4, H)

    update = pl.pallas_call(
        _update_kernel,
        grid=(grid,),
        in_specs=[row_spec, row_spec, row_spec, w_spec, b_spec],
        out_specs=row_spec,
        out_shape=out_rows,
    )
    for _ in range(DEPTH - 1):
        g = _gather_sum(msg, a2a4, H)
        msg = update(x, g, cpart, W_ha, b_h2)

    g = _gather_sum(msg, a2a4, H)
    ah = pl.pallas_call(
        _final_kernel,
        grid=(grid,),
        in_specs=[row_spec, row_spec, w_spec, b_spec],
        out_specs=row_spec,
        out_shape=out_rows,
    )(fo, g, W_ob, b_o2)

    ah3 = ah[:N].reshape(NMOL, APM, H)
    mv3 = pl.pallas_call(
        _readout_kernel,
        grid=(NMOL,),
        in_specs=[pl.BlockSpec((1, APM, H), lambda i: (i, 0, 0))],
        out_specs=pl.BlockSpec((1, 1, H), lambda i: (i, 0, 0)),
        out_shape=jax.ShapeDtypeStruct((NMOL, 1, H), f32),
    )(ah3)
    mol_vecs = mv3.reshape(NMOL, H)

    out = pl.pallas_call(
        _ffn_kernel,
        grid=(1,),
        in_specs=[
            pl.BlockSpec((NMOL, H), _full),
            pl.BlockSpec((H, 1), _full),
            pl.BlockSpec(memory_space=pltpu.SMEM),
        ],
        out_specs=pl.BlockSpec((NMOL, 1), _full),
        out_shape=jax.ShapeDtypeStruct((NMOL, 1), f32),
    )(mol_vecs, W_f, b_f)
    return out


# speedup vs baseline: 1.2002x; 1.1469x over previous
import functools

import jax
import jax.numpy as jnp
from jax import lax
from jax.experimental import pallas as pl
from jax.experimental.pallas import tpu as pltpu
from jax.experimental.pallas import tpu_sc as plsc

N = 10000
NB = 32
H = 128
BF = 16
NMOL = 100
APM = 100
DEPTH = 3

NC = 2
NS = 16
NW = NC * NS
NP = 10240
AW = NP // NW
SA = 8
STEPS = AW // SA
CH = (SA * NB) // 128


def _reduce_rows(rows_v, out_v, d):
    nch = d // 16
    for a in range(SA):
        def body(j, accs, a=a):
            return tuple(
                accs[c] + rows_v[a * NB + j, pl.ds(c * 16, 16)]
                for c in range(nch))
        accs = lax.fori_loop(
            0, NB, body, tuple(jnp.zeros((16,), jnp.float32)
                               for _ in range(nch)))
        for c in range(nch):
            out_v[a, pl.ds(c * 16, 16)] = accs[c]


def _gather_sum_kernel(table_hbm, idx_hbm, out_hbm, idx_v,
                       rows0, rows1, out0, out1, gsem0, gsem1, osem0, osem1):
    c = lax.axis_index("c")
    s = lax.axis_index("s")
    w = c * NS + s
    d = out0.shape[1]

    pltpu.sync_copy(idx_hbm.at[w], idx_v)

    def gather(t, rows, gsem):
        for j in range(CH):
            pltpu.async_copy(table_hbm.at[idx_v.at[t, j]],
                             rows.at[pl.ds(j * 128, 128)], gsem)

    def wait_gather(t, rows, gsem):
        for j in range(CH):
            pltpu.make_async_copy(table_hbm.at[idx_v.at[t, j]],
                                  rows.at[pl.ds(j * 128, 128)], gsem).wait()

    def out_slice(t):
        return out_hbm.at[pl.ds(w * AW + t * SA, SA)]

    gather(0, rows0, gsem0)
    gather(1, rows1, gsem1)

    @pl.loop(0, STEPS // 2)
    def _(i):
        t0 = 2 * i
        for (t, rows, out_v, gsem, osem) in (
                (t0, rows0, out0, gsem0, osem0),
                (t0 + 1, rows1, out1, gsem1, osem1)):
            wait_gather(t, rows, gsem)

            @pl.when(t >= 2)
            def _():
                pltpu.make_async_copy(out_v, out_slice(t - 2), osem).wait()

            _reduce_rows(rows, out_v, d)
            pltpu.async_copy(out_v, out_slice(t), osem)

            @pl.when(t + 2 < STEPS)
            def _():
                gather(t + 2, rows, gsem)

    pltpu.make_async_copy(out0, out_slice(STEPS - 2), osem0).wait()
    pltpu.make_async_copy(out1, out_slice(STEPS - 1), osem1).wait()


def _gather_sum(table, idx4, d):
    mesh = plsc.VectorSubcoreMesh(core_axis_name="c", subcore_axis_name="s")
    return pl.kernel(
        _gather_sum_kernel,
        out_type=jax.ShapeDtypeStruct((NP, d), jnp.float32),
        mesh=mesh,
        scratch_types=[
            pltpu.VMEM((STEPS, CH, 128), jnp.int32),
            pltpu.VMEM((SA * NB, d), jnp.float32),
            pltpu.VMEM((SA * NB, d), jnp.float32),
            pltpu.VMEM((SA, d), jnp.float32),
            pltpu.VMEM((SA, d), jnp.float32),
            pltpu.SemaphoreType.DMA,
            pltpu.SemaphoreType.DMA,
            pltpu.SemaphoreType.DMA,
            pltpu.SemaphoreType.DMA,
        ],
    )(table, idx4)


BM = 256


def _pre_kernel(a_ref, wi_ref, bi_ref, wo_ref, x_ref, msg_ref, fo_ref):
    a = a_ref[...]
    x = jnp.dot(a, wi_ref[...], preferred_element_type=jnp.float32) + bi_ref[...]
    x_ref[...] = x
    msg_ref[...] = jnp.maximum(x, 0.0)
    fo_ref[...] = jnp.dot(a, wo_ref[...], preferred_element_type=jnp.float32)


def _bondproj_kernel(fb_ref, wb_ref, out_ref):
    out_ref[...] = jnp.dot(fb_ref[...], wb_ref[...],
                           preferred_element_type=jnp.float32)


def _update_kernel(x_ref, g_ref, cpart_ref, wa_ref, bh_ref, out_ref):
    m = jnp.dot(g_ref[...], wa_ref[...], preferred_element_type=jnp.float32)
    out_ref[...] = jnp.maximum(x_ref[...] + m + cpart_ref[...] + bh_ref[...],
                               0.0)


def _final_kernel(fo_ref, g_ref, wo2_ref, bo_ref, out_ref):
    m = jnp.dot(g_ref[...], wo2_ref[...], preferred_element_type=jnp.float32)
    out_ref[...] = jnp.maximum(fo_ref[...] + m + bo_ref[...], 0.0)


def _readout_kernel(ah_ref, out_ref):
    out_ref[0] = jnp.max(ah_ref[0], axis=0, keepdims=True)


def _ffn_kernel(mv_ref, wf_ref, bf_ref, out_ref):
    out_ref[...] = (jnp.dot(mv_ref[...], wf_ref[...],
                            preferred_element_type=jnp.float32) + bf_ref[0])


def _row_block(i):
    return (i, 0)


def _full(i):
    return (0, 0)


@jax.jit
def kernel(f_atoms, f_bonds, a2a, a2b, W_i, b_i, W_h, b_h, W_o, b_o, W_f, b_f):
    f32 = jnp.float32
    f_atoms = f_atoms.astype(f32)
    f_bonds = f_bonds.astype(f32)

    f_pad = jnp.pad(f_atoms, ((0, NP - N), (0, 0)))
    a2a_i = jnp.pad(a2a.astype(jnp.int32), ((0, NP - N), (0, 0)))
    a2b_i = jnp.pad(a2b.astype(jnp.int32), ((0, NP - N), (0, 0)))
    a2a4 = a2a_i.reshape(NW, STEPS, CH, 128)
    a2b4 = a2b_i.reshape(NW, STEPS, CH, 128)
    W_ha = W_h[:H]
    W_hb = W_h[H:]
    W_ot = W_o[:H]
    W_ob = W_o[H:]
    b_i2 = b_i.reshape(1, H)
    b_h2 = b_h.reshape(1, H)
    b_o2 = b_o.reshape(1, H)

    grid = NP // BM
    row_spec = pl.BlockSpec((BM, H), _row_block)
    w_spec = pl.BlockSpec((H, H), _full)
    wb_spec = pl.BlockSpec((BF, H), _full)
    b_spec = pl.BlockSpec((1, H), _full)
    out_rows = jax.ShapeDtypeStruct((NP, H), f32)

    x, msg, fo = pl.pallas_call(
        _pre_kernel,
        grid=(grid,),
        in_specs=[row_spec, w_spec, b_spec, w_spec],
        out_specs=[row_spec, row_spec, row_spec],
        out_shape=[out_rows, out_rows, out_rows],
    )(f_pad, W_i, b_i2, W_ot)

    BMB = 1280
    fbp = pl.pallas_call(
        _bondproj_kernel,
        grid=(f_bonds.shape[0] // BMB,),
        in_specs=[pl.BlockSpec((BMB, BF), _row_block), wb_spec],
        out_specs=pl.BlockSpec((BMB, H), _row_block),
        out_shape=jax.ShapeDtypeStruct((f_bonds.shape[0], H), f32),
    )(f_bonds, W_hb)
    cpart = _gather_sum(fbp, a2b4, H)

    update = pl.pallas_call(
        _update_kernel,
        grid=(grid,),
        in_specs=[row_spec, row_spec, row_spec, w_spec, b_spec],
        out_specs=row_spec,
        out_shape=out_rows,
    )
    for _ in range(DEPTH - 1):
        g = _gather_sum(msg, a2a4, H)
        msg = update(x, g, cpart, W_ha, b_h2)

    g = _gather_sum(msg, a2a4, H)
    ah = pl.pallas_call(
        _final_kernel,
        grid=(grid,),
        in_specs=[row_spec, row_spec, w_spec, b_spec],
        out_specs=row_spec,
        out_shape=out_rows,
    )(fo, g, W_ob, b_o2)

    ah3 = ah[:N].reshape(NMOL, APM, H)
    mv3 = pl.pallas_call(
        _readout_kernel,
        grid=(NMOL,),
        in_specs=[pl.BlockSpec((1, APM, H), lambda i: (i, 0, 0))],
        out_specs=pl.BlockSpec((1, 1, H), lambda i: (i, 0, 0)),
        out_shape=jax.ShapeDtypeStruct((NMOL, 1, H), f32),
    )(ah3)
    mol_vecs = mv3.reshape(NMOL, H)

    out = pl.pallas_call(
        _ffn_kernel,
        grid=(1,),
        in_specs=[
            pl.BlockSpec((NMOL, H), _full),
            pl.BlockSpec((H, 1), _full),
            pl.BlockSpec(memory_space=pltpu.SMEM),
        ],
        out_specs=pl.BlockSpec((NMOL, 1), _full),
        out_shape=jax.ShapeDtypeStruct((NMOL, 1), f32),
    )(mol_vecs, W_f, b_f)
    return out


# speedup vs baseline: 1.9959x; 1.6630x over previous
import functools

import jax
import jax.numpy as jnp
from jax import lax
from jax.experimental import pallas as pl
from jax.experimental.pallas import tpu as pltpu
from jax.experimental.pallas import tpu_sc as plsc

N = 10000
NB = 32
H = 128
BF = 16
NMOL = 100
APM = 100
DEPTH = 3

NC = 2
NS = 16
NW = NC * NS
NP = 10240
AW = NP // NW
SA = 8
STEPS = AW // SA
CH = (SA * NB) // 128


def _reduce_rows(rows_v, out_v, d):
    nch = d // 16
    for a in range(SA):
        def body(j, accs, a=a):
            return tuple(
                accs[c] + rows_v[a * NB + j, pl.ds(c * 16, 16)]
                for c in range(nch))
        accs = lax.fori_loop(
            0, NB, body, tuple(jnp.zeros((16,), jnp.float32)
                               for _ in range(nch)))
        for c in range(nch):
            out_v[a, pl.ds(c * 16, 16)] = accs[c]


def _pipeline(gather, wait_gather, reduce, out_slice, steps,
              rows0, rows1, out0, out1, gsem0, gsem1, osem0, osem1):
    gather(0, rows0, gsem0)
    gather(1, rows1, gsem1)

    @pl.loop(0, steps // 2)
    def _(i):
        t0 = 2 * i
        for (t, rows, out_v, gsem, osem) in (
                (t0, rows0, out0, gsem0, osem0),
                (t0 + 1, rows1, out1, gsem1, osem1)):
            wait_gather(t, rows, gsem)

            @pl.when(t >= 2)
            def _():
                pltpu.make_async_copy(out_v, out_slice(t - 2), osem).wait()

            reduce(rows, out_v)
            pltpu.async_copy(out_v, out_slice(t), osem)

            @pl.when(t + 2 < steps)
            def _():
                gather(t + 2, rows, gsem)

    pltpu.make_async_copy(out0, out_slice(steps - 2), osem0).wait()
    pltpu.make_async_copy(out1, out_slice(steps - 1), osem1).wait()


def _gather_sum_hbm_kernel(table_hbm, idx_hbm, out_hbm, idx_v,
                           rows0, rows1, out0, out1,
                           gsem0, gsem1, osem0, osem1):
    c = lax.axis_index("c")
    s = lax.axis_index("s")
    w = c * NS + s
    d = out0.shape[1]

    pltpu.sync_copy(idx_hbm.at[w], idx_v)

    def gather(t, rows, gsem):
        for j in range(CH):
            pltpu.async_copy(table_hbm.at[idx_v.at[t, j]],
                             rows.at[pl.ds(j * 128, 128)], gsem)

    def wait_gather(t, rows, gsem):
        for j in range(CH):
            pltpu.make_async_copy(table_hbm.at[idx_v.at[t, j]],
                                  rows.at[pl.ds(j * 128, 128)], gsem).wait()

    def out_slice(t):
        return out_hbm.at[pl.ds(w * AW + t * SA, SA)]

    _pipeline(gather, wait_gather,
              lambda rows, out_v: _reduce_rows(rows, out_v, d),
              out_slice, STEPS,
              rows0, rows1, out0, out1, gsem0, gsem1, osem0, osem1)


def _gather_sum(table, idx4, d):
    mesh = plsc.VectorSubcoreMesh(core_axis_name="c", subcore_axis_name="s")
    return pl.kernel(
        _gather_sum_hbm_kernel,
        out_type=jax.ShapeDtypeStruct((NP, d), jnp.float32),
        mesh=mesh,
        scratch_types=[
            pltpu.VMEM((STEPS, CH, 128), jnp.int32),
            pltpu.VMEM((SA * NB, d), jnp.float32),
            pltpu.VMEM((SA * NB, d), jnp.float32),
            pltpu.VMEM((SA, d), jnp.float32),
            pltpu.VMEM((SA, d), jnp.float32),
            pltpu.SemaphoreType.DMA,
            pltpu.SemaphoreType.DMA,
            pltpu.SemaphoreType.DMA,
            pltpu.SemaphoreType.DMA,
        ],
    )(table, idx4)


HALF = NP // 2
AW2 = NP // NS
STEPS2 = AW2 // SA


def _gather_sum_split_kernel(table_hbm, idx_hbm, out_hbm, idx_v,
                             rows0, rows1, out0, out1,
                             gsem0, gsem1, osem0, osem1, spm):
    c = lax.axis_index("c")
    s = lax.axis_index("s")
    d = out0.shape[1]

    pltpu.sync_copy(idx_hbm.at[c, s], idx_v)

    share = HALF // NS
    pltpu.sync_copy(table_hbm.at[pl.ds(c * HALF + s * share, share)],
                    spm.at[pl.ds(s * share, share)])

    @pl.when(s == 0)
    def _():
        pltpu.sync_copy(table_hbm.at[pl.ds(NP, 8)], spm.at[pl.ds(HALF, 8)])

    plsc.subcore_barrier()

    def gather(t, rows, gsem):
        for j in range(CH):
            pltpu.async_copy(spm.at[idx_v.at[t, j]],
                             rows.at[pl.ds(j * 128, 128)], gsem)

    def wait_gather(t, rows, gsem):
        for j in range(CH):
            pltpu.make_async_copy(spm.at[idx_v.at[t, j]],
                                  rows.at[pl.ds(j * 128, 128)], gsem).wait()

    def out_slice(t):
        return out_hbm.at[c].at[pl.ds(s * AW2 + t * SA, SA)]

    _pipeline(gather, wait_gather,
              lambda rows, out_v: _reduce_rows(rows, out_v, d),
              out_slice, STEPS2,
              rows0, rows1, out0, out1, gsem0, gsem1, osem0, osem1)


def _gather_sum_split(table_z, idx2, d):
    mesh = plsc.VectorSubcoreMesh(core_axis_name="c", subcore_axis_name="s")
    return pl.kernel(
        _gather_sum_split_kernel,
        out_type=jax.ShapeDtypeStruct((2, NP, d), jnp.float32),
        mesh=mesh,
        scratch_types=[
            pltpu.VMEM((STEPS2, CH, 128), jnp.int32),
            pltpu.VMEM((SA * NB, d), jnp.float32),
            pltpu.VMEM((SA * NB, d), jnp.float32),
            pltpu.VMEM((SA, d), jnp.float32),
            pltpu.VMEM((SA, d), jnp.float32),
            pltpu.SemaphoreType.DMA,
            pltpu.SemaphoreType.DMA,
            pltpu.SemaphoreType.DMA,
            pltpu.SemaphoreType.DMA,
            pltpu.VMEM_SHARED((HALF + 8, d), jnp.float32),
        ],
    )(table_z, idx2)


BM = 256


def _pre_kernel(a_ref, wi_ref, bi_ref, wo_ref, x_ref, msg_ref, fo_ref):
    a = a_ref[...]
    x = jnp.dot(a, wi_ref[...], preferred_element_type=jnp.float32) + bi_ref[...]
    x_ref[...] = x
    msg_ref[...] = jnp.maximum(x, 0.0)
    fo_ref[...] = jnp.dot(a, wo_ref[...], preferred_element_type=jnp.float32)


def _bondproj_kernel(fb_ref, wb_ref, out_ref):
    out_ref[...] = jnp.dot(fb_ref[...], wb_ref[...],
                           preferred_element_type=jnp.float32)


def _update_kernel(x_ref, gl_ref, gh_ref, cpart_ref, wa_ref, bh_ref, out_ref):
    g = gl_ref[...] + gh_ref[...]
    m = jnp.dot(g, wa_ref[...], preferred_element_type=jnp.float32)
    out_ref[...] = jnp.maximum(x_ref[...] + m + cpart_ref[...] + bh_ref[...],
                               0.0)


def _final_kernel(fo_ref, gl_ref, gh_ref, wo2_ref, bo_ref, out_ref):
    g = gl_ref[...] + gh_ref[...]
    m = jnp.dot(g, wo2_ref[...], preferred_element_type=jnp.float32)
    out_ref[...] = jnp.maximum(fo_ref[...] + m + bo_ref[...], 0.0)


def _readout_kernel(ah_ref, out_ref):
    out_ref[0] = jnp.max(ah_ref[0], axis=0, keepdims=True)


def _ffn_kernel(mv_ref, wf_ref, bf_ref, out_ref):
    out_ref[...] = (jnp.dot(mv_ref[...], wf_ref[...],
                            preferred_element_type=jnp.float32) + bf_ref[0])


def _row_block(i):
    return (i, 0)


def _full(i):
    return (0, 0)


@jax.jit
def kernel(f_atoms, f_bonds, a2a, a2b, W_i, b_i, W_h, b_h, W_o, b_o, W_f, b_f):
    f32 = jnp.float32
    f_atoms = f_atoms.astype(f32)
    f_bonds = f_bonds.astype(f32)

    f_pad = jnp.pad(f_atoms, ((0, NP - N), (0, 0)))
    a2a_i = jnp.pad(a2a.astype(jnp.int32), ((0, NP - N), (0, 0)))
    a2b_i = jnp.pad(a2b.astype(jnp.int32), ((0, NP - N), (0, 0)))
    a2b4 = a2b_i.reshape(NW, STEPS, CH, 128)
    aL = jnp.where(a2a_i < HALF, a2a_i, HALF)
    aH = jnp.where(a2a_i >= HALF, a2a_i - HALF, HALF)
    idx2 = jnp.stack([aL.reshape(NS, STEPS2, CH, 128),
                      aH.reshape(NS, STEPS2, CH, 128)])
    W_ha = W_h[:H]
    W_hb = W_h[H:]
    W_ot = W_o[:H]
    W_ob = W_o[H:]
    b_i2 = b_i.reshape(1, H)
    b_h2 = b_h.reshape(1, H)
    b_o2 = b_o.reshape(1, H)

    grid = NP // BM
    row_spec = pl.BlockSpec((BM, H), _row_block)
    w_spec = pl.BlockSpec((H, H), _full)
    wb_spec = pl.BlockSpec((BF, H), _full)
    b_spec = pl.BlockSpec((1, H), _full)
    out_rows = jax.ShapeDtypeStruct((NP, H), f32)
    out_rows_bf = jax.ShapeDtypeStruct((NP, H), jnp.bfloat16)

    x, msg, fo = pl.pallas_call(
        _pre_kernel,
        grid=(grid,),
        in_specs=[row_spec, w_spec, b_spec, w_spec],
        out_specs=[row_spec, row_spec, row_spec],
        out_shape=[out_rows, out_rows, out_rows],
    )(f_pad, W_i, b_i2, W_ot)

    BMB = 1280
    fbp = pl.pallas_call(
        _bondproj_kernel,
        grid=(f_bonds.shape[0] // BMB,),
        in_specs=[pl.BlockSpec((BMB, BF), _row_block), wb_spec],
        out_specs=pl.BlockSpec((BMB, H), _row_block),
        out_shape=jax.ShapeDtypeStruct((f_bonds.shape[0], H), f32),
    )(f_bonds, W_hb)
    cpart = _gather_sum(fbp, a2b4, H)

    update = pl.pallas_call(
        _update_kernel,
        grid=(grid,),
        in_specs=[row_spec, row_spec, row_spec, row_spec, w_spec, b_spec],
        out_specs=row_spec,
        out_shape=out_rows,
    )
    for _ in range(DEPTH - 1):
        g2 = _gather_sum_split(jnp.pad(msg, ((0, 8), (0, 0))), idx2, H)
        msg = update(x, g2[0], g2[1], cpart, W_ha, b_h2)

    g2 = _gather_sum_split(jnp.pad(msg, ((0, 8), (0, 0))), idx2, H)
    ah = pl.pallas_call(
        _final_kernel,
        grid=(grid,),
        in_specs=[row_spec, row_spec, row_spec, w_spec, b_spec],
        out_specs=row_spec,
        out_shape=out_rows,
    )(fo, g2[0], g2[1], W_ob, b_o2)

    ah3 = ah[:N].reshape(NMOL, APM, H)
    mv3 = pl.pallas_call(
        _readout_kernel,
        grid=(NMOL,),
        in_specs=[pl.BlockSpec((1, APM, H), lambda i: (i, 0, 0))],
        out_specs=pl.BlockSpec((1, 1, H), lambda i: (i, 0, 0)),
        out_shape=jax.ShapeDtypeStruct((NMOL, 1, H), f32),
    )(ah3)
    mol_vecs = mv3.reshape(NMOL, H)

    out = pl.pallas_call(
        _ffn_kernel,
        grid=(1,),
        in_specs=[
            pl.BlockSpec((NMOL, H), _full),
            pl.BlockSpec((H, 1), _full),
            pl.BlockSpec(memory_space=pltpu.SMEM),
        ],
        out_specs=pl.BlockSpec((NMOL, 1), _full),
        out_shape=jax.ShapeDtypeStruct((NMOL, 1), f32),
    )(mol_vecs, W_f, b_f)
    return out
